# baseline (speedup 1.0000x reference)
import jax
import jax.numpy as jnp
from jax.experimental import pallas as pl
from jax.experimental.pallas import tpu as pltpu

_C = 1024
_OUT = 2048
_NQ = 4
_KQ = _C // _NQ
_NQW = _OUT // _NQ

_CHUNK = 2048
_NRB = 3
_NOB = 4


def _chunk_sizes(rows_per_core):
    sizes = []
    rem = rows_per_core
    for s in (128, 384, 512, 1024):
        if rem <= s:
            break
        sizes.append(s)
        rem -= s
    while rem > 0:
        s = min(_CHUNK, rem)
        sizes.append(s)
        rem -= s
    return sizes


def _manual_kernel(x_hbm, w_ref, o_hbm, rbuf, obuf, rsem, wsem):
    core = pl.program_id(0)
    rows_per_core = x_hbm.shape[0] // 2
    base = core * rows_per_core
    sizes = _chunk_sizes(rows_per_core)
    offs = [0]
    for s in sizes:
        offs.append(offs[-1] + s)
    nch = len(sizes)

    def read_cp(k):
        return pltpu.make_async_copy(
            x_hbm.at[pl.ds(base + offs[k], sizes[k]), :],
            rbuf.at[k % _NRB, pl.ds(0, sizes[k])],
            rsem.at[k % _NRB],
        )

    def write_cp(k):
        return pltpu.make_async_copy(
            obuf.at[k % _NOB, pl.ds(0, sizes[k])],
            o_hbm.at[pl.ds(base + offs[k], sizes[k]), :],
            wsem.at[k % _NOB],
        )

    for k in range(min(_NRB - 1, nch)):
        read_cp(k).start()

    for k in range(nch):
        if k + _NRB - 1 < nch:
            read_cp(k + _NRB - 1).start()
        read_cp(k).wait()
        if k >= _NOB:
            write_cp(k - _NOB).wait()
        xc = rbuf[k % _NRB, 0:sizes[k]]
        for q in range(_NQ):
            mask = (xc[:, _KQ * q:_KQ * (q + 1)] == 1).astype(w_ref.dtype)
            wq = w_ref[_KQ * q:_KQ * (q + 1), _NQW * q:_NQW * (q + 1)]
            obuf[k % _NOB, 0:sizes[k], _NQW * q:_NQW * (q + 1)] = jnp.dot(
                mask, wq, preferred_element_type=jnp.float32
            ).astype(jnp.bfloat16)
        write_cp(k).start()

    for k in range(max(0, nch - _NOB), nch):
        write_cp(k).wait()


def _emb_manual(x, w):
    B = x.shape[0]
    return pl.pallas_call(
        _manual_kernel,
        out_shape=jax.ShapeDtypeStruct((B, _OUT), jnp.bfloat16),
        grid=(2,),
        in_specs=[
            pl.BlockSpec(memory_space=pl.ANY),
            pl.BlockSpec((_C, _OUT), lambda i: (0, 0)),
        ],
        out_specs=pl.BlockSpec(memory_space=pl.ANY),
        scratch_shapes=[
            pltpu.VMEM((_NRB, _CHUNK, _C), jnp.int8),
            pltpu.VMEM((_NOB, _CHUNK, _OUT), jnp.bfloat16),
            pltpu.SemaphoreType.DMA((_NRB,)),
            pltpu.SemaphoreType.DMA((_NOB,)),
        ],
        compiler_params=pltpu.CompilerParams(
            dimension_semantics=("parallel",),
            vmem_limit_bytes=60 << 20,
        ),
    )(x, w)


def _quad_kernel(x_ref, w_ref, o_ref):
    for q in range(_NQ):
        mask = (x_ref[:, _KQ * q:_KQ * (q + 1)] == 1).astype(w_ref.dtype)
        wq = w_ref[_KQ * q:_KQ * (q + 1), _NQW * q:_NQW * (q + 1)]
        o_ref[:, _NQW * q:_NQW * (q + 1)] = jnp.dot(
            mask, wq, preferred_element_type=jnp.float32
        ).astype(o_ref.dtype)


def _emb_blocked(x, w, bb):
    B = x.shape[0]
    B_pad = ((B + bb - 1) // bb) * bb
    x_p = x if B_pad == B else jnp.pad(x, ((0, B_pad - B), (0, 0)))
    out = pl.pallas_call(
        _quad_kernel,
        out_shape=jax.ShapeDtypeStruct((B_pad, _OUT), jnp.bfloat16),
        grid=(B_pad // bb,),
        in_specs=[
            pl.BlockSpec((bb, _C), lambda i: (i, 0)),
            pl.BlockSpec((_C, _OUT), lambda i: (0, 0)),
        ],
        out_specs=pl.BlockSpec((bb, _OUT), lambda i: (i, 0)),
        compiler_params=pltpu.CompilerParams(
            dimension_semantics=("parallel",),
            vmem_limit_bytes=60 << 20,
        ),
    )(x_p, w)
    return out[:B] if B_pad != B else out


def kernel(x, w):
    squeeze = False
    if x.ndim < 2:
        x = x[None, :]
        squeeze = True
    B, C = x.shape
    assert C == _C and w.shape == (_C, _OUT)

    if B % (2 * _CHUNK) == 0 and B >= 4 * _CHUNK:
        out = _emb_manual(x, w)
    else:
        bb = min(4096, max(((B + 7) // 8) * 8, 8))
        out = _emb_blocked(x, w, bb)

    if squeeze:
        out = out[0]
    return out


# speedup vs baseline: 1.0059x; 1.0059x over previous
import jax
import jax.numpy as jnp
from jax.experimental import pallas as pl
from jax.experimental.pallas import tpu as pltpu

_C = 1024
_OUT = 2048
_NQ = 4
_KQ = _C // _NQ
_NQW = _OUT // _NQ

_CHUNK = 2048
_NRB = 3
_NOB = 4


def _chunk_sizes(rows_per_core):
    sizes = []
    rem = rows_per_core
    for s in (256, 768, 1024):
        if rem <= s:
            break
        sizes.append(s)
        rem -= s
    while rem > 0:
        s = min(_CHUNK, rem)
        sizes.append(s)
        rem -= s
    return sizes


def _manual_kernel(x_hbm, w_ref, o_hbm, rbuf, obuf, rsem, wsem):
    core = pl.program_id(0)
    rows_per_core = x_hbm.shape[0] // 2
    base = core * rows_per_core
    sizes = _chunk_sizes(rows_per_core)
    offs = [0]
    for s in sizes:
        offs.append(offs[-1] + s)
    nch = len(sizes)

    def read_cp(k):
        return pltpu.make_async_copy(
            x_hbm.at[pl.ds(base + offs[k], sizes[k]), :],
            rbuf.at[k % _NRB, pl.ds(0, sizes[k])],
            rsem.at[k % _NRB],
        )

    def write_cp(k):
        return pltpu.make_async_copy(
            obuf.at[k % _NOB, pl.ds(0, sizes[k])],
            o_hbm.at[pl.ds(base + offs[k], sizes[k]), :],
            wsem.at[k % _NOB],
        )

    for k in range(min(_NRB - 1, nch)):
        read_cp(k).start()

    for k in range(nch):
        if k + _NRB - 1 < nch:
            read_cp(k + _NRB - 1).start()
        read_cp(k).wait()
        if k >= _NOB:
            write_cp(k - _NOB).wait()
        xc = rbuf[k % _NRB, 0:sizes[k]]
        for q in range(_NQ):
            mask = (xc[:, _KQ * q:_KQ * (q + 1)] == 1).astype(w_ref.dtype)
            wq = w_ref[_KQ * q:_KQ * (q + 1), _NQW * q:_NQW * (q + 1)]
            obuf[k % _NOB, 0:sizes[k], _NQW * q:_NQW * (q + 1)] = jnp.dot(
                mask, wq, preferred_element_type=jnp.float32
            ).astype(jnp.bfloat16)
        write_cp(k).start()

    for k in range(max(0, nch - _NOB), nch):
        write_cp(k).wait()


def _emb_manual(x, w):
    B = x.shape[0]
    return pl.pallas_call(
        _manual_kernel,
        out_shape=jax.ShapeDtypeStruct((B, _OUT), jnp.bfloat16),
        grid=(2,),
        in_specs=[
            pl.BlockSpec(memory_space=pl.ANY),
            pl.BlockSpec((_C, _OUT), lambda i: (0, 0)),
        ],
        out_specs=pl.BlockSpec(memory_space=pl.ANY),
        scratch_shapes=[
            pltpu.VMEM((_NRB, _CHUNK, _C), jnp.int8),
            pltpu.VMEM((_NOB, _CHUNK, _OUT), jnp.bfloat16),
            pltpu.SemaphoreType.DMA((_NRB,)),
            pltpu.SemaphoreType.DMA((_NOB,)),
        ],
        compiler_params=pltpu.CompilerParams(
            dimension_semantics=("parallel",),
            vmem_limit_bytes=60 << 20,
        ),
    )(x, w)


def _quad_kernel(x_ref, w_ref, o_ref):
    for q in range(_NQ):
        mask = (x_ref[:, _KQ * q:_KQ * (q + 1)] == 1).astype(w_ref.dtype)
        wq = w_ref[_KQ * q:_KQ * (q + 1), _NQW * q:_NQW * (q + 1)]
        o_ref[:, _NQW * q:_NQW * (q + 1)] = jnp.dot(
            mask, wq, preferred_element_type=jnp.float32
        ).astype(o_ref.dtype)


def _emb_blocked(x, w, bb):
    B = x.shape[0]
    B_pad = ((B + bb - 1) // bb) * bb
    x_p = x if B_pad == B else jnp.pad(x, ((0, B_pad - B), (0, 0)))
    out = pl.pallas_call(
        _quad_kernel,
        out_shape=jax.ShapeDtypeStruct((B_pad, _OUT), jnp.bfloat16),
        grid=(B_pad // bb,),
        in_specs=[
            pl.BlockSpec((bb, _C), lambda i: (i, 0)),
            pl.BlockSpec((_C, _OUT), lambda i: (0, 0)),
        ],
        out_specs=pl.BlockSpec((bb, _OUT), lambda i: (i, 0)),
        compiler_params=pltpu.CompilerParams(
            dimension_semantics=("parallel",),
            vmem_limit_bytes=60 << 20,
        ),
    )(x_p, w)
    return out[:B] if B_pad != B else out


def kernel(x, w):
    squeeze = False
    if x.ndim < 2:
        x = x[None, :]
        squeeze = True
    B, C = x.shape
    assert C == _C and w.shape == (_C, _OUT)

    if B % (2 * _CHUNK) == 0 and B >= 4 * _CHUNK:
        out = _emb_manual(x, w)
    else:
        bb = min(4096, max(((B + 7) // 8) * 8, 8))
        out = _emb_blocked(x, w, bb)

    if squeeze:
        out = out[0]
    return out


# speedup vs baseline: 1.0340x; 1.0280x over previous
import jax
import jax.numpy as jnp
from jax.experimental import pallas as pl
from jax.experimental.pallas import tpu as pltpu

_C = 1024
_OUT = 2048
_NQ = 4
_KQ = _C // _NQ
_NQW = _OUT // _NQ

_CHUNK = 2048
_NRB = 4
_NOB = 4


def _chunk_sizes(rows_per_core):
    sizes = []
    rem = rows_per_core
    for s in (256, 768, 1024):
        if rem <= s:
            break
        sizes.append(s)
        rem -= s
    while rem > 0:
        s = min(_CHUNK, rem)
        sizes.append(s)
        rem -= s
    return sizes


def _manual_kernel(x_hbm, w_ref, o_hbm, rbuf, obuf, rsem, wsem):
    core = pl.program_id(0)
    rows_per_core = x_hbm.shape[0] // 2
    base = core * rows_per_core
    sizes = _chunk_sizes(rows_per_core)
    offs = [0]
    for s in sizes:
        offs.append(offs[-1] + s)
    nch = len(sizes)

    def read_cp(k):
        return pltpu.make_async_copy(
            x_hbm.at[pl.ds(base + offs[k], sizes[k]), :],
            rbuf.at[k % _NRB, pl.ds(0, sizes[k])],
            rsem.at[k % _NRB],
        )

    def write_cp(k):
        return pltpu.make_async_copy(
            obuf.at[k % _NOB, pl.ds(0, sizes[k])],
            o_hbm.at[pl.ds(base + offs[k], sizes[k]), :],
            wsem.at[k % _NOB],
        )

    for k in range(min(_NRB - 1, nch)):
        read_cp(k).start()

    for k in range(nch):
        if k + _NRB - 1 < nch:
            read_cp(k + _NRB - 1).start()
        read_cp(k).wait()
        if k >= _NOB:
            write_cp(k - _NOB).wait()
        xc = rbuf[k % _NRB, 0:sizes[k]]
        for q in range(_NQ):
            mask = (xc[:, _KQ * q:_KQ * (q + 1)] == 1).astype(w_ref.dtype)
            wq = w_ref[_KQ * q:_KQ * (q + 1), _NQW * q:_NQW * (q + 1)]
            obuf[k % _NOB, 0:sizes[k], _NQW * q:_NQW * (q + 1)] = jnp.dot(
                mask, wq, preferred_element_type=jnp.float32
            ).astype(jnp.bfloat16)
        write_cp(k).start()

    for k in range(max(0, nch - _NOB), nch):
        write_cp(k).wait()


def _emb_manual(x, w):
    B = x.shape[0]
    return pl.pallas_call(
        _manual_kernel,
        out_shape=jax.ShapeDtypeStruct((B, _OUT), jnp.bfloat16),
        grid=(2,),
        in_specs=[
            pl.BlockSpec(memory_space=pl.ANY),
            pl.BlockSpec((_C, _OUT), lambda i: (0, 0)),
        ],
        out_specs=pl.BlockSpec(memory_space=pl.ANY),
        scratch_shapes=[
            pltpu.VMEM((_NRB, _CHUNK, _C), jnp.int8),
            pltpu.VMEM((_NOB, _CHUNK, _OUT), jnp.bfloat16),
            pltpu.SemaphoreType.DMA((_NRB,)),
            pltpu.SemaphoreType.DMA((_NOB,)),
        ],
        compiler_params=pltpu.CompilerParams(
            dimension_semantics=("parallel",),
            vmem_limit_bytes=60 << 20,
        ),
    )(x, w)


def _quad_kernel(x_ref, w_ref, o_ref):
    for q in range(_NQ):
        mask = (x_ref[:, _KQ * q:_KQ * (q + 1)] == 1).astype(w_ref.dtype)
        wq = w_ref[_KQ * q:_KQ * (q + 1), _NQW * q:_NQW * (q + 1)]
        o_ref[:, _NQW * q:_NQW * (q + 1)] = jnp.dot(
            mask, wq, preferred_element_type=jnp.float32
        ).astype(o_ref.dtype)


def _emb_blocked(x, w, bb):
    B = x.shape[0]
    B_pad = ((B + bb - 1) // bb) * bb
    x_p = x if B_pad == B else jnp.pad(x, ((0, B_pad - B), (0, 0)))
    out = pl.pallas_call(
        _quad_kernel,
        out_shape=jax.ShapeDtypeStruct((B_pad, _OUT), jnp.bfloat16),
        grid=(B_pad // bb,),
        in_specs=[
            pl.BlockSpec((bb, _C), lambda i: (i, 0)),
            pl.BlockSpec((_C, _OUT), lambda i: (0, 0)),
        ],
        out_specs=pl.BlockSpec((bb, _OUT), lambda i: (i, 0)),
        compiler_params=pltpu.CompilerParams(
            dimension_semantics=("parallel",),
            vmem_limit_bytes=60 << 20,
        ),
    )(x_p, w)
    return out[:B] if B_pad != B else out


def kernel(x, w):
    squeeze = False
    if x.ndim < 2:
        x = x[None, :]
        squeeze = True
    B, C = x.shape
    assert C == _C and w.shape == (_C, _OUT)

    if B % (2 * _CHUNK) == 0 and B >= 4 * _CHUNK:
        out = _emb_manual(x, w)
    else:
        bb = min(4096, max(((B + 7) // 8) * 8, 8))
        out = _emb_blocked(x, w, bb)

    if squeeze:
        out = out[0]
    return out
